# baseline (speedup 1.0000x reference)
import jax
import jax.numpy as jnp
from jax import lax
from jax.experimental import pallas as pl
from jax.experimental.pallas import tpu as pltpu
from jax.experimental.pallas import tpu_sc as plsc

N = 10000
E = 320000
D = 128
DW = 160
NS = 16
CH = 128
NCHUNK = E // CH
CPT = NCHUNK // NS
EXTRA = NCHUNK - CPT * NS
IB = 6
NBLK = CPT // IB
RPT = N // NS


def _leaky(x):
    return jnp.where(x >= 0, x, 0.01 * x)


def _proj_body(ft_ref, fc_ref, w_ref, b_ref, o_ref):
    br = ft_ref.shape[0]
    lane = lax.broadcasted_iota(jnp.int32, (br, DW - D), 1)
    cpad = jnp.where(lane == 0, 1.0, 0.0).astype(jnp.bfloat16)
    acc_t = jnp.dot(ft_ref[...], w_ref[0],
                    preferred_element_type=jnp.float32)
    o_ref[0, :, 0:D] = _leaky(acc_t + b_ref[0]).astype(jnp.bfloat16)
    o_ref[0, :, D:DW] = cpad
    acc_c = jnp.dot(fc_ref[...], w_ref[1],
                    preferred_element_type=jnp.float32)
    o_ref[1, :, 0:D] = _leaky(acc_c + b_ref[1]).astype(jnp.bfloat16)
    o_ref[1, :, D:DW] = cpad


def _tc_project(feat_table, feat_column, Ws, bs):
    BR = 1000
    grid = (N // BR,)
    return pl.pallas_call(
        _proj_body,
        grid=grid,
        in_specs=[
            pl.BlockSpec((BR, D), lambda i: (i, 0)),
            pl.BlockSpec((BR, D), lambda i: (i, 0)),
            pl.BlockSpec((2, D, D), lambda i: (0, 0, 0)),
            pl.BlockSpec((2, 1, D), lambda i: (0, 0, 0)),
        ],
        out_specs=pl.BlockSpec((2, BR, DW), lambda i: (0, i, 0)),
        out_shape=jax.ShapeDtypeStruct((2, N, DW), jnp.bfloat16),
    )(feat_table, feat_column, Ws, bs)


def _sc_agg_body(wh_hbm, e_hbm, sums_hbm,
                 sidx0, sidx1, didx0, didx1,
                 rows0, rows1, rows2, rows3,
                 accum_sh,
                 g0, g1, g2, g3, s0, s1, s2, s3, isem):
    c = lax.axis_index("c")
    s = lax.axis_index("s")
    rows = (rows0, rows1, rows2, rows3)
    gsem = (g0, g1, g2, g3)
    ssem = (s0, s1, s2, s3)
    sidx = (sidx0, sidx1)
    didx = (didx0, didx1)

    zb16 = jnp.zeros((32,), jnp.bfloat16)

    def _zrow(i, _):
        for q in range(DW // 32):
            rows0[i, pl.ds(q * 32, 32)] = zb16
        return 0
    lax.fori_loop(0, CH, _zrow, 0)

    row0 = s * RPT
    for q in range(4):
        pltpu.sync_copy(rows0, accum_sh.at[pl.ds(row0 + q * CH, CH)])
    pltpu.sync_copy(rows0.at[pl.ds(0, RPT - 4 * CH)],
                    accum_sh.at[pl.ds(row0 + 4 * CH, RPT - 4 * CH)])
    plsc.subcore_barrier()

    chunk0 = s * CPT

    def _gather(iref, k):
        pltpu.async_copy(wh_hbm.at[c].at[iref], rows[k], gsem[k])

    def _wait_gather(k):
        pltpu.make_async_copy(wh_hbm.at[c].at[sidx0.at[0]], rows[k],
                              gsem[k]).wait()

    def _scatter(dref, k):
        pltpu.async_copy(rows[k], accum_sh.at[dref], ssem[k], add=True)

    def _wait_scatter(k):
        pltpu.make_async_copy(rows[k], accum_sh.at[didx0.at[0]],
                              ssem[k]).wait()

    def _stage_idx(t, bb, sync):
        base = chunk0 + t * IB
        pltpu.async_copy(e_hbm.at[c, 0, pl.ds(base, IB)], sidx[bb], isem)
        pltpu.async_copy(e_hbm.at[c, 1, pl.ds(base, IB)], didx[bb], isem)
        if sync:
            _wait_idx(bb)

    def _wait_idx(bb):
        pltpu.make_async_copy(e_hbm.at[0, 0, pl.ds(0, IB)], sidx[bb],
                              isem).wait()
        pltpu.make_async_copy(e_hbm.at[0, 1, pl.ds(0, IB)], didx[bb],
                              isem).wait()

    _stage_idx(0, 0, True)
    for k in range(4):
        _gather(sidx0.at[k], k)

    def _do_block(t, bb):
        @pl.when(t + 1 < NBLK)
        def _():
            _stage_idx(t + 1, 1 - bb, False)

        for jl in range(IB):
            k = (2 * bb + jl) % 4
            _wait_gather(k)
            _scatter(didx[bb].at[jl], k)
            _wait_scatter(k)
            nxt = jl + 4
            if jl == IB - 4:
                @pl.when(t + 1 < NBLK)
                def _():
                    _wait_idx(1 - bb)
            if nxt < IB:
                _gather(sidx[bb].at[nxt], k)
            else:
                @pl.when(t + 1 < NBLK)
                def _():
                    _gather(sidx[1 - bb].at[nxt - IB], k)

    def _pair(u, _):
        _do_block(2 * u, 0)
        _do_block(2 * u + 1, 1)
        return 0

    lax.fori_loop(0, NBLK // 2, _pair, 0)

    @pl.when(s < EXTRA)
    def _():
        base = CPT * NS + s
        pltpu.sync_copy(e_hbm.at[c, 0, pl.ds(base, 1)],
                        sidx0.at[pl.ds(0, 1)])
        pltpu.sync_copy(e_hbm.at[c, 1, pl.ds(base, 1)],
                        didx0.at[pl.ds(0, 1)])
        pltpu.async_copy(wh_hbm.at[c].at[sidx0.at[0]], rows0, g0).wait()
        pltpu.sync_copy(rows0, accum_sh.at[didx0.at[0]], add=True)

    plsc.subcore_barrier()

    pltpu.sync_copy(accum_sh.at[pl.ds(row0, RPT)],
                    sums_hbm.at[c, pl.ds(row0, RPT)])


def _sc_aggregate(wh, edges):
    mesh = plsc.VectorSubcoreMesh(core_axis_name="c", subcore_axis_name="s")
    k = pl.kernel(
        _sc_agg_body,
        out_type=jax.ShapeDtypeStruct((2, N, DW), jnp.bfloat16),
        mesh=mesh,
        scratch_types=[
            pltpu.VMEM((IB, CH), jnp.int32),
            pltpu.VMEM((IB, CH), jnp.int32),
            pltpu.VMEM((IB, CH), jnp.int32),
            pltpu.VMEM((IB, CH), jnp.int32),
            pltpu.VMEM((CH, DW), jnp.bfloat16),
            pltpu.VMEM((CH, DW), jnp.bfloat16),
            pltpu.VMEM((CH, DW), jnp.bfloat16),
            pltpu.VMEM((CH, DW), jnp.bfloat16),
            pltpu.VMEM_SHARED((N, DW), jnp.bfloat16),
            pltpu.SemaphoreType.DMA,
            pltpu.SemaphoreType.DMA,
            pltpu.SemaphoreType.DMA,
            pltpu.SemaphoreType.DMA,
            pltpu.SemaphoreType.DMA,
            pltpu.SemaphoreType.DMA,
            pltpu.SemaphoreType.DMA,
            pltpu.SemaphoreType.DMA,
            pltpu.SemaphoreType.DMA,
        ],
        compiler_params=pltpu.CompilerParams(use_tc_tiling_on_sc=False),
    )
    return k(wh, edges)


def _final_body(s0_ref, s1_ref, ft_ref, fc_ref, w_ref, b_ref,
                ot_ref, oc_ref):
    w = w_ref[...]
    b = b_ref[...]
    s1 = s1_ref[0].astype(jnp.float32)
    c1 = jnp.maximum(s1[:, D:D + 1], 1.0)
    h1 = s1[:, 0:D] / c1
    ot_ref[...] = _leaky(
        jnp.dot(h1, w, preferred_element_type=jnp.float32) + b) + ft_ref[...]
    s0 = s0_ref[0].astype(jnp.float32)
    c0 = jnp.maximum(s0[:, D:D + 1], 1.0)
    h0 = s0[:, 0:D] / c0
    oc_ref[...] = _leaky(
        jnp.dot(h0, w, preferred_element_type=jnp.float32) + b) + fc_ref[...]


def _tc_final(sums, feat_table, feat_column, W_h, b_h):
    BR = 1000
    grid = (N // BR,)
    out = pl.pallas_call(
        _final_body,
        grid=grid,
        in_specs=[
            pl.BlockSpec((1, BR, DW), lambda i: (0, i, 0)),
            pl.BlockSpec((1, BR, DW), lambda i: (1, i, 0)),
            pl.BlockSpec((BR, D), lambda i: (i, 0)),
            pl.BlockSpec((BR, D), lambda i: (i, 0)),
            pl.BlockSpec((D, D), lambda i: (0, 0)),
            pl.BlockSpec((1, D), lambda i: (0, 0)),
        ],
        out_specs=[
            pl.BlockSpec((BR, D), lambda i: (i, 0)),
            pl.BlockSpec((BR, D), lambda i: (i, 0)),
        ],
        out_shape=[
            jax.ShapeDtypeStruct((N, D), jnp.float32),
            jax.ShapeDtypeStruct((N, D), jnp.float32),
        ],
    )(sums, sums, feat_table, feat_column, W_h, b_h)
    return out


def kernel(feat_table, feat_column, edge_t2c, edge_c2t,
           W_t2c, b_t2c, W_c2t, b_c2t, W_h, b_h):
    Ws = jnp.stack([W_t2c, W_c2t])
    bs = jnp.stack([b_t2c, b_c2t]).reshape(2, 1, D)
    wh = _tc_project(feat_table, feat_column, Ws, bs)

    edges = jnp.concatenate([edge_t2c.astype(jnp.int32),
                             edge_c2t.astype(jnp.int32)],
                            axis=0).reshape(2, 2, NCHUNK, CH)
    sums = _sc_aggregate(wh, edges)

    out_table, out_column = _tc_final(sums, feat_table, feat_column,
                                      W_h, b_h.reshape(1, D))
    return out_table, out_column


# speedup vs baseline: 1.2450x; 1.2450x over previous
import jax
import jax.numpy as jnp
from jax import lax
from jax.experimental import pallas as pl
from jax.experimental.pallas import tpu as pltpu
from jax.experimental.pallas import tpu_sc as plsc

N = 10000
E = 320000
D = 128
NS = 16
CH = 128
NCHUNK = E // CH
CPT = NCHUNK // NS
EXTRA = NCHUNK - CPT * NS
IB = 6
NBLK = CPT // IB
RPT = N // NS
NPAD = 640
STRIP = NPAD // NS


def _leaky(x):
    return jnp.where(x >= 0, x, 0.01 * x)


def _proj_body(ft_ref, fc_ref, w_ref, b_ref, o_ref):
    acc_t = jnp.dot(ft_ref[...], w_ref[0],
                    preferred_element_type=jnp.float32)
    o_ref[0] = _leaky(acc_t + b_ref[0]).astype(jnp.bfloat16)
    acc_c = jnp.dot(fc_ref[...], w_ref[1],
                    preferred_element_type=jnp.float32)
    o_ref[1] = _leaky(acc_c + b_ref[1]).astype(jnp.bfloat16)


def _tc_project(feat_table, feat_column, Ws, bs):
    BR = 1000
    grid = (N // BR,)
    return pl.pallas_call(
        _proj_body,
        grid=grid,
        in_specs=[
            pl.BlockSpec((BR, D), lambda i: (i, 0)),
            pl.BlockSpec((BR, D), lambda i: (i, 0)),
            pl.BlockSpec((2, D, D), lambda i: (0, 0, 0)),
            pl.BlockSpec((2, 1, D), lambda i: (0, 0, 0)),
        ],
        out_specs=pl.BlockSpec((2, BR, D), lambda i: (0, i, 0)),
        out_shape=jax.ShapeDtypeStruct((2, N, D), jnp.bfloat16),
    )(feat_table, feat_column, Ws, bs)


def _sc_agg_body(wh_hbm, e_hbm, sums_hbm, cnts_hbm,
                 sidx0, sidx1, didx0, didx1,
                 rows0, rows1, rows2, rows3,
                 cnt_local, mbuf, outbuf,
                 accum_sh, callg_sh,
                 g0, g1, g2, g3, s0, s1, s2, s3, isem):
    c = lax.axis_index("c")
    s = lax.axis_index("s")
    rows = (rows0, rows1, rows2, rows3)
    gsem = (g0, g1, g2, g3)
    ssem = (s0, s1, s2, s3)
    sidx = (sidx0, sidx1)
    didx = (didx0, didx1)

    zb16 = jnp.zeros((32,), jnp.bfloat16)
    zi32 = jnp.zeros((16,), jnp.int32)

    def _zrow(i, _):
        for q in range(D // 32):
            rows0[i, pl.ds(q * 32, 32)] = zb16
        return 0
    lax.fori_loop(0, CH, _zrow, 0)

    row0 = s * RPT
    for q in range(4):
        pltpu.sync_copy(rows0, accum_sh.at[pl.ds(row0 + q * CH, CH)])
    pltpu.sync_copy(rows0.at[pl.ds(0, RPT - 4 * CH)],
                    accum_sh.at[pl.ds(row0 + 4 * CH, RPT - 4 * CH)])

    def _zcnt(i, _):
        cnt_local[i, :] = zi32
        return 0
    lax.fori_loop(0, NPAD, _zcnt, 0)
    plsc.subcore_barrier()

    chunk0 = s * CPT

    def _gather(iref, k):
        pltpu.async_copy(wh_hbm.at[c].at[iref], rows[k], gsem[k])

    def _wait_gather(k):
        pltpu.make_async_copy(wh_hbm.at[c].at[sidx0.at[0]], rows[k],
                              gsem[k]).wait()

    def _scatter(dref, k):
        pltpu.async_copy(rows[k], accum_sh.at[dref], ssem[k], add=True)

    def _wait_scatter(k):
        pltpu.make_async_copy(rows[k], accum_sh.at[didx0.at[0]],
                              ssem[k]).wait()

    def _stage_idx(t, bb, sync):
        base = chunk0 + t * IB
        pltpu.async_copy(e_hbm.at[c, 0, pl.ds(base, IB)], sidx[bb], isem)
        pltpu.async_copy(e_hbm.at[c, 1, pl.ds(base, IB)], didx[bb], isem)
        if sync:
            _wait_idx(bb)

    def _wait_idx(bb):
        pltpu.make_async_copy(e_hbm.at[0, 0, pl.ds(0, IB)], sidx[bb],
                              isem).wait()
        pltpu.make_async_copy(e_hbm.at[0, 1, pl.ds(0, IB)], didx[bb],
                              isem).wait()

    def _hist(dref, jl):
        for q in range(CH // 16):
            idx = dref[jl, pl.ds(q * 16, 16)]
            run, last = plsc.scan_count(idx)
            r = lax.shift_right_logical(idx, 4)
            col = lax.bitwise_and(idx, 15)
            plsc.addupdate_scatter(cnt_local, [r, col], run, mask=last)

    _stage_idx(0, 0, True)
    for k in range(4):
        _gather(sidx0.at[k], k)

    def _do_block(t, bb):
        @pl.when(t + 1 < NBLK)
        def _():
            _stage_idx(t + 1, 1 - bb, False)

        for jl in range(IB):
            k = (2 * bb + jl) % 4
            _wait_gather(k)
            _scatter(didx[bb].at[jl], k)
            _hist(didx[bb], jl)
            _wait_scatter(k)
            nxt = jl + 4
            if jl == IB - 4:
                @pl.when(t + 1 < NBLK)
                def _():
                    _wait_idx(1 - bb)
            if nxt < IB:
                _gather(sidx[bb].at[nxt], k)
            else:
                @pl.when(t + 1 < NBLK)
                def _():
                    _gather(sidx[1 - bb].at[nxt - IB], k)

    def _pair(u, _):
        _do_block(2 * u, 0)
        _do_block(2 * u + 1, 1)
        return 0

    lax.fori_loop(0, NBLK // 2, _pair, 0)

    @pl.when(s < EXTRA)
    def _():
        base = CPT * NS + s
        pltpu.sync_copy(e_hbm.at[c, 0, pl.ds(base, 1)],
                        sidx0.at[pl.ds(0, 1)])
        pltpu.sync_copy(e_hbm.at[c, 1, pl.ds(base, 1)],
                        didx0.at[pl.ds(0, 1)])
        pltpu.async_copy(wh_hbm.at[c].at[sidx0.at[0]], rows0, g0).wait()
        pltpu.sync_copy(rows0, accum_sh.at[didx0.at[0]], add=True)
        _hist(didx0, 0)

    pltpu.sync_copy(cnt_local, callg_sh.at[s])
    plsc.subcore_barrier()
    for r in range(NS):
        pltpu.sync_copy(callg_sh.at[r, pl.ds(s * STRIP, STRIP)],
                        mbuf.at[r])
    iota16 = lax.iota(jnp.int32, 16)

    def _merge(i, _):
        acc = mbuf[0, i, :]
        for r in range(1, NS):
            acc = acc + mbuf[r, i, :]
        plsc.store_scatter(outbuf, [16 * i + iota16, iota16 * 0], acc)
        return 0

    lax.fori_loop(0, STRIP, _merge, 0)
    pltpu.sync_copy(outbuf, cnts_hbm.at[c, pl.ds(s * (16 * STRIP),
                                                 16 * STRIP)])

    pltpu.sync_copy(accum_sh.at[pl.ds(row0, RPT)],
                    sums_hbm.at[c, pl.ds(row0, RPT)])


def _sc_aggregate(wh, edges):
    mesh = plsc.VectorSubcoreMesh(core_axis_name="c", subcore_axis_name="s")
    k = pl.kernel(
        _sc_agg_body,
        out_type=[
            jax.ShapeDtypeStruct((2, N, D), jnp.bfloat16),
            jax.ShapeDtypeStruct((2, NS * NPAD, 16), jnp.int32),
        ],
        mesh=mesh,
        scratch_types=[
            pltpu.VMEM((IB, CH), jnp.int32),
            pltpu.VMEM((IB, CH), jnp.int32),
            pltpu.VMEM((IB, CH), jnp.int32),
            pltpu.VMEM((IB, CH), jnp.int32),
            pltpu.VMEM((CH, D), jnp.bfloat16),
            pltpu.VMEM((CH, D), jnp.bfloat16),
            pltpu.VMEM((CH, D), jnp.bfloat16),
            pltpu.VMEM((CH, D), jnp.bfloat16),
            pltpu.VMEM((NPAD, 16), jnp.int32),
            pltpu.VMEM((NS, STRIP, 16), jnp.int32),
            pltpu.VMEM((16 * STRIP, 16), jnp.int32),
            pltpu.VMEM_SHARED((N, D), jnp.bfloat16),
            pltpu.VMEM_SHARED((NS, NPAD, 16), jnp.int32),
            pltpu.SemaphoreType.DMA,
            pltpu.SemaphoreType.DMA,
            pltpu.SemaphoreType.DMA,
            pltpu.SemaphoreType.DMA,
            pltpu.SemaphoreType.DMA,
            pltpu.SemaphoreType.DMA,
            pltpu.SemaphoreType.DMA,
            pltpu.SemaphoreType.DMA,
            pltpu.SemaphoreType.DMA,
        ],
        compiler_params=pltpu.CompilerParams(use_tc_tiling_on_sc=False,
                                             needs_layout_passes=False),
    )
    return k(wh, edges)


def _final_body(s0_ref, s1_ref, c0_ref, c1_ref, ft_ref, fc_ref,
                w_ref, b_ref, ot_ref, oc_ref):
    w = w_ref[...]
    b = b_ref[...]
    c1 = jnp.maximum(c1_ref[0][:, 0:1].astype(jnp.float32), 1.0)
    h1 = s1_ref[0].astype(jnp.float32) / c1
    ot_ref[...] = _leaky(
        jnp.dot(h1, w, preferred_element_type=jnp.float32) + b) + ft_ref[...]
    c0 = jnp.maximum(c0_ref[0][:, 0:1].astype(jnp.float32), 1.0)
    h0 = s0_ref[0].astype(jnp.float32) / c0
    oc_ref[...] = _leaky(
        jnp.dot(h0, w, preferred_element_type=jnp.float32) + b) + fc_ref[...]


def _tc_final(sums, cnts, feat_table, feat_column, W_h, b_h):
    BR = 1000
    grid = (N // BR,)
    out = pl.pallas_call(
        _final_body,
        grid=grid,
        in_specs=[
            pl.BlockSpec((1, BR, D), lambda i: (0, i, 0)),
            pl.BlockSpec((1, BR, D), lambda i: (1, i, 0)),
            pl.BlockSpec((1, BR, 16), lambda i: (0, i, 0)),
            pl.BlockSpec((1, BR, 16), lambda i: (1, i, 0)),
            pl.BlockSpec((BR, D), lambda i: (i, 0)),
            pl.BlockSpec((BR, D), lambda i: (i, 0)),
            pl.BlockSpec((D, D), lambda i: (0, 0)),
            pl.BlockSpec((1, D), lambda i: (0, 0)),
        ],
        out_specs=[
            pl.BlockSpec((BR, D), lambda i: (i, 0)),
            pl.BlockSpec((BR, D), lambda i: (i, 0)),
        ],
        out_shape=[
            jax.ShapeDtypeStruct((N, D), jnp.float32),
            jax.ShapeDtypeStruct((N, D), jnp.float32),
        ],
    )(sums, sums, cnts, cnts, feat_table, feat_column, W_h, b_h)
    return out


def kernel(feat_table, feat_column, edge_t2c, edge_c2t,
           W_t2c, b_t2c, W_c2t, b_c2t, W_h, b_h):
    Ws = jnp.stack([W_t2c, W_c2t])
    bs = jnp.stack([b_t2c, b_c2t]).reshape(2, 1, D)
    wh = _tc_project(feat_table, feat_column, Ws, bs)

    edges = jnp.concatenate([edge_t2c.astype(jnp.int32),
                             edge_c2t.astype(jnp.int32)],
                            axis=0).reshape(2, 2, NCHUNK, CH)
    sums, cnts = _sc_aggregate(wh, edges)

    out_table, out_column = _tc_final(sums, cnts, feat_table, feat_column,
                                      W_h, b_h.reshape(1, D))
    return out_table, out_column
